# Initial kernel scaffold; baseline (speedup 1.0000x reference)
import jax
import jax.numpy as jnp
from jax.experimental import pallas as pl


def kernel(acids, mask, idx):
    raise NotImplementedError("write your pallas kernel here")


# speedup vs baseline: 27.8766x; 27.8766x over previous
import functools

import jax
import jax.numpy as jnp
from jax import lax
from jax.experimental import pallas as pl
from jax.experimental.pallas import tpu as pltpu
from jax.experimental.pallas import tpu_sc as plsc

L = 48
W = 7
B = 4
D = 4 * (L - 1) + 1
NP = 2 * L - 1
NPP = 96
WROW = 352

NCORES = 2
NSUB = 16
NW = NCORES * NSUB
WIN_PER_W = (B * L) // NW
SUB_PER_B = L // WIN_PER_W
NCHUNK = NPP // 16


def _sc_windows(idx_pack, vals, starts_rep):
    mesh = plsc.VectorSubcoreMesh(
        core_axis_name="c", subcore_axis_name="s",
        num_cores=NCORES, num_subcores=NSUB,
    )

    @functools.partial(
        pl.kernel,
        mesh=mesh,
        out_type=jax.ShapeDtypeStruct((B * L, WROW), jnp.float32),
        compiler_params=pltpu.CompilerParams(needs_layout_passes=False),
        scratch_types=[
            pltpu.VMEM((3, NPP), jnp.int32),
            pltpu.VMEM((NPP,), jnp.float32),
            pltpu.VMEM((3, 16), jnp.int32),
            pltpu.VMEM((WROW,), jnp.float32),
        ],
    )
    def body(idx_hbm, vals_hbm, starts_hbm, out_hbm, pts_v, val_v, st_v, win_v):
        wid = lax.axis_index("s") * NCORES + lax.axis_index("c")
        b = wid // SUB_PER_B
        i0 = (wid % SUB_PER_B) * WIN_PER_W

        pltpu.sync_copy(idx_hbm.at[b], pts_v)
        pltpu.sync_copy(vals_hbm.at[b], val_v)

        zeros = jnp.zeros((16,), jnp.float32)
        for w in range(WIN_PER_W):
            win = wid * WIN_PER_W + w
            pltpu.sync_copy(starts_hbm.at[win], st_v)
            sx = st_v[0, :]
            sy = st_v[1, :]
            sz = st_v[2, :]

            for k in range(WROW // 16):
                win_v[pl.ds(k * 16, 16)] = zeros

            for t in range(NCHUNK):
                sl = pl.ds(t * 16, 16)
                dx = pts_v[0, sl] - sx
                dy = pts_v[1, sl] - sy
                dz = pts_v[2, sl] - sz
                m = (
                    (dx >= 0) & (dx < W)
                    & (dy >= 0) & (dy < W)
                    & (dz >= 0) & (dz < W)
                )
                off = dx * (W * W) + dy * W + dz
                off = jnp.where(m, off, 0)
                plsc.addupdate_scatter(win_v, [off], val_v[sl], mask=m)

            pltpu.sync_copy(win_v, out_hbm.at[wid * WIN_PER_W + w])

    return body(idx_pack, vals, starts_rep)


def kernel(acids, mask, idx):
    idx2 = 2 * (idx.astype(jnp.int32) + (L - 1))
    inter_idx = (idx2[:, :-1, :] + idx2[:, 1:, :]) >> 1
    inter_vals = acids[:, :-1] + acids[:, 1:] + 1.0

    combined_idx = jnp.concatenate([idx2, inter_idx], axis=1)
    combined_vals = jnp.concatenate(
        [acids * mask, inter_vals * mask[:, 1:]], axis=1)

    pad_i = jnp.zeros((B, NPP - NP, 3), jnp.int32)
    pad_v = jnp.zeros((B, NPP - NP), jnp.float32)
    idx_pack = jnp.concatenate([combined_idx, pad_i], axis=1)
    idx_pack = idx_pack.transpose(0, 2, 1)
    vals = jnp.concatenate([combined_vals, pad_v], axis=1)

    starts = jnp.clip(idx2[:, :L, :] - W // 2, 0, D - W)
    starts_rep = jnp.broadcast_to(
        starts.reshape(B * L, 3, 1), (B * L, 3, 16)
    ).astype(jnp.int32)

    rows = _sc_windows(idx_pack, vals, starts_rep)
    out = rows[:, : W * W * W].reshape(B, L, W, W, W)
    return out[..., None]


# speedup vs baseline: 33.4346x; 1.1994x over previous
import functools

import jax
import jax.numpy as jnp
from jax import lax
from jax.experimental import pallas as pl
from jax.experimental.pallas import tpu as pltpu
from jax.experimental.pallas import tpu_sc as plsc

L = 48
W = 7
B = 4
D = 4 * (L - 1) + 1
WVOL = W * W * W
WROW = 352

NCORES = 2
NSUB = 16
NW = NCORES * NSUB
WIN_PER_W = (B * L) // NW
SUB_PER_B = L // WIN_PER_W
ACC = WIN_PER_W * WROW


def _sc_windows(acids, mask, idx_flat):
    mesh = plsc.VectorSubcoreMesh(
        core_axis_name="c", subcore_axis_name="s",
        num_cores=NCORES, num_subcores=NSUB,
    )

    @functools.partial(
        pl.kernel,
        mesh=mesh,
        out_type=jax.ShapeDtypeStruct((NW, ACC), jnp.float32),
        compiler_params=pltpu.CompilerParams(
            needs_layout_passes=False, use_tc_tiling_on_sc=False),
        scratch_types=[
            pltpu.VMEM((L + 16,), jnp.float32),
            pltpu.VMEM((L + 16,), jnp.float32),
            pltpu.VMEM((3 * L,), jnp.int32),
            pltpu.VMEM((ACC,), jnp.float32),
            pltpu.SemaphoreType.DMA,
            pltpu.SemaphoreType.DMA,
            pltpu.SemaphoreType.DMA,
        ],
    )
    def body(acids_hbm, mask_hbm, idx_hbm, out_hbm,
             ac_v, mk_v, ix_v, win_v, sem_a, sem_m, sem_i):
        wid = lax.axis_index("s") * NCORES + lax.axis_index("c")
        b = wid // SUB_PER_B
        i0 = (wid % SUB_PER_B) * WIN_PER_W

        cp_a = pltpu.async_copy(acids_hbm.at[b], ac_v.at[pl.ds(0, L)], sem_a)
        cp_m = pltpu.async_copy(mask_hbm.at[b], mk_v.at[pl.ds(0, L)], sem_m)
        cp_i = pltpu.async_copy(idx_hbm.at[b], ix_v, sem_i)

        zeros = jnp.zeros((16,), jnp.float32)
        for k in range(ACC // 16):
            win_v[pl.ds(k * 16, 16)] = zeros

        cp_a.wait()
        cp_m.wait()
        cp_i.wait()

        lane = jnp.arange(16, dtype=jnp.int32)

        chunks = []
        for t in range(3):
            pid3 = (t * 16) * 3 + lane * 3
            cx = 2 * (plsc.load_gather(ix_v, [pid3]) + (L - 1))
            cy = 2 * (plsc.load_gather(ix_v, [pid3 + 1]) + (L - 1))
            cz = 2 * (plsc.load_gather(ix_v, [pid3 + 2]) + (L - 1))
            v = ac_v[pl.ds(t * 16, 16)] * mk_v[pl.ds(t * 16, 16)]
            valid = lane < 16
            chunks.append((cx, cy, cz, v, valid))
        for t in range(3):
            mid = t * 16 + lane
            valid = mid < (L - 1)
            midc = jnp.where(valid, mid, L - 2)
            pid3 = midc * 3
            cx = (plsc.load_gather(ix_v, [pid3])
                  + plsc.load_gather(ix_v, [pid3 + 3]) + 2 * (L - 1))
            cy = (plsc.load_gather(ix_v, [pid3 + 1])
                  + plsc.load_gather(ix_v, [pid3 + 4]) + 2 * (L - 1))
            cz = (plsc.load_gather(ix_v, [pid3 + 2])
                  + plsc.load_gather(ix_v, [pid3 + 5]) + 2 * (L - 1))
            v = (ac_v[pl.ds(t * 16, 16)] + ac_v[pl.ds(t * 16 + 1, 16)]
                 + 1.0) * mk_v[pl.ds(t * 16 + 1, 16)]
            chunks.append((cx, cy, cz, v, valid))

        for w in range(WIN_PER_W):
            i3 = (i0 + w) * 3
            sx = jnp.clip(
                2 * (plsc.load_gather(ix_v, [jnp.full((16,), i3, jnp.int32)])
                     + (L - 1)) - W // 2, 0, D - W)
            sy = jnp.clip(
                2 * (plsc.load_gather(ix_v, [jnp.full((16,), i3 + 1, jnp.int32)])
                     + (L - 1)) - W // 2, 0, D - W)
            sz = jnp.clip(
                2 * (plsc.load_gather(ix_v, [jnp.full((16,), i3 + 2, jnp.int32)])
                     + (L - 1)) - W // 2, 0, D - W)
            for (cx, cy, cz, v, valid) in chunks:
                dx = cx - sx
                dy = cy - sy
                dz = cz - sz
                m = (
                    (dx >= 0) & (dx < W)
                    & (dy >= 0) & (dy < W)
                    & (dz >= 0) & (dz < W)
                    & valid
                )
                off = dx * (W * W) + dy * W + dz + w * WROW
                off = jnp.where(m, off, w * WROW)
                plsc.addupdate_scatter(win_v, [off], v, mask=m)

        pltpu.sync_copy(win_v, out_hbm.at[wid])

    return body(acids, mask, idx_flat)


def kernel(acids, mask, idx):
    idx_flat = idx.astype(jnp.int32).reshape(B, 3 * L)
    rows = _sc_windows(acids, mask, idx_flat)
    rows = rows.reshape(B * L, WROW)
    out = rows[:, :WVOL].reshape(B, L, W, W, W)
    return out[..., None]
